# Initial kernel scaffold; baseline (speedup 1.0000x reference)
import jax
import jax.numpy as jnp
from jax.experimental import pallas as pl


def kernel(h, edge_index, params):
    raise NotImplementedError("write your pallas kernel here")


# speedup vs baseline: 7.2900x; 7.2900x over previous
import functools
import jax
import jax.numpy as jnp
from jax import lax
from jax.experimental import pallas as pl
from jax.experimental.pallas import tpu as pltpu
from jax.experimental.pallas import tpu_sc as plsc

N = 10000
E = 160000
B = 128
EP = 163840
R = EP // B
NTILE = 16
NCORE = 2
CH = 128
WTILES = 10
WROWS = N // WTILES
ZR = 40
RPT = R // NTILE
RPT32 = R // (NTILE * NCORE)
NP = 10008
ND = 10240
TAB = 80128
BN = 1000
GRID = N // BN

_SC_PARAMS = pltpu.CompilerParams(needs_layout_passes=False)


def _zero_rows(zbuf):
  @pl.loop(0, zbuf.shape[0])
  def _(r):
    for j in range(zbuf.shape[1] // 16):
      zbuf[r, pl.ds(j * 16, 16)] = jnp.zeros((16,), jnp.float32)


def _coop_zero(s, acc, zbuf):
  @pl.when(s < WTILES)
  def _():
    @pl.loop(0, WROWS // ZR)
    def _(i):
      pltpu.sync_copy(zbuf, acc.at[pl.ds(s * WROWS + i * ZR, ZR)])


def _seg_sum_body(nchunk, table, src2d, dst2d, out,
                  idx_src, idx_dst, rows, zbuf, acc, sem):
  c = lax.axis_index("c")
  s = lax.axis_index("s")
  _zero_rows(zbuf)
  pltpu.sync_copy(src2d.at[pl.ds(s * RPT, RPT)], idx_src)
  pltpu.sync_copy(dst2d.at[pl.ds(s * RPT, RPT)], idx_dst)

  for k in range(nchunk // NCORE):
    chunk = 2 * k + c
    _coop_zero(s, acc, zbuf)
    plsc.subcore_barrier()

    @pl.loop(0, RPT)
    def _(j):
      pltpu.async_copy(table.at[idx_src.at[j], pl.ds(chunk * CH, CH)],
                       rows, sem).wait()
      pltpu.sync_copy(rows, acc.at[idx_dst.at[j]], add=True)
    plsc.subcore_barrier()

    @pl.when(s < WTILES)
    def _():
      pltpu.sync_copy(acc.at[pl.ds(s * WROWS, WROWS)],
                      out.at[pl.ds(s * WROWS, WROWS), pl.ds(chunk * CH, CH)])
    plsc.subcore_barrier()
  if (nchunk // NCORE) % 2 == 1:
    plsc.subcore_barrier()


def sc_segment_sum(table, src2d, dst2d):
  n, d = table.shape
  nchunk = d // CH
  mesh = plsc.VectorSubcoreMesh(core_axis_name="c", subcore_axis_name="s")
  fn = pl.kernel(
      functools.partial(_seg_sum_body, nchunk),
      out_type=jax.ShapeDtypeStruct((n, d), jnp.float32),
      mesh=mesh,
      scratch_types=[
          pltpu.VMEM((RPT, B), jnp.int32),
          pltpu.VMEM((RPT, B), jnp.int32),
          pltpu.VMEM((B, CH), jnp.float32),
          pltpu.VMEM((ZR, CH), jnp.float32),
          pltpu.VMEM_SHARED((NP, CH), jnp.float32),
          pltpu.SemaphoreType.DMA,
      ],
  )
  return fn(table, src2d, dst2d)


def _sc2d_zero(ref):
  lanes = lax.iota(jnp.int32, 16)
  zero = jnp.zeros((16,), jnp.float32)
  @pl.loop(0, ref.shape[0])
  def _(r):
    rv = jnp.full((16,), r, jnp.int32)
    for j in range(ref.shape[1] // 16):
      plsc.store_scatter(ref, [rv, lanes + j * 16], zero)


def _degree_body(dst2d, out, idx_dst, deg_v, idxr, sacc):
  c = lax.axis_index("c")
  s = lax.axis_index("s")
  base = (c * NTILE + s) * RPT32
  pltpu.sync_copy(dst2d.at[pl.ds(base, RPT32)], idx_dst)

  _sc2d_zero(deg_v)
  lanes = lax.iota(jnp.int32, 16)
  for g in range((ND // CH) // 16):
    idxr[pl.ds(g * 16, 16)] = lanes + g * 16

  ones = jnp.ones((16,), jnp.float32)
  @pl.loop(0, RPT32)
  def _(j):
    jv = jnp.full((16,), j, jnp.int32)
    for g in range(B // 16):
      v = plsc.load_gather(idx_dst, [jv, lanes + g * 16])
      plsc.addupdate_scatter(deg_v, [v >> 7, v & 127], ones)

  @pl.when(s == 0)
  def _():
    pltpu.sync_copy(deg_v, sacc)
  plsc.subcore_barrier()
  @pl.when(s > 0)
  def _():
    pltpu.sync_copy(deg_v, sacc.at[idxr], add=True)
  plsc.subcore_barrier()

  @pl.when(s < WTILES)
  def _():
    pltpu.sync_copy(sacc.at[pl.ds(s * 8, 8)], out.at[c, pl.ds(s * 8, 8)])


def sc_degree(dst2d):
  mesh = plsc.VectorSubcoreMesh(core_axis_name="c", subcore_axis_name="s")
  fn = pl.kernel(
      _degree_body,
      out_type=jax.ShapeDtypeStruct((NCORE, ND // CH, CH), jnp.float32),
      mesh=mesh,
      compiler_params=_SC_PARAMS,
      scratch_types=[
          pltpu.VMEM((RPT32, B), jnp.int32),
          pltpu.VMEM((ND // CH, CH), jnp.float32),
          pltpu.VMEM((ND // CH,), jnp.int32),
          pltpu.VMEM_SHARED((ND // CH, CH), jnp.float32),
      ],
  )
  return fn(dst2d)


def _gat_edge_body(tab128, src2d, dst2d, shift_hbm, ee_out, esum_out,
                   idx_src, idx_dst, rows_s, rows_d, ee_flat, shift_v,
                   acc, sem):
  c = lax.axis_index("c")
  s = lax.axis_index("s")
  w = c * NTILE + s
  base = w * RPT32
  pltpu.sync_copy(shift_hbm, shift_v)
  pltpu.sync_copy(src2d.at[pl.ds(base, RPT32)], idx_src)
  pltpu.sync_copy(dst2d.at[pl.ds(base, RPT32)], idx_dst)

  _sc2d_zero(rows_d)
  @pl.when(s < WTILES)
  def _():
    @pl.loop(0, WROWS // ZR)
    def _(i):
      pltpu.sync_copy(rows_d.at[pl.ds(0, ZR)],
                      acc.at[pl.ds(s * WROWS + i * ZR, ZR)])
  plsc.subcore_barrier()

  sh = shift_v[pl.ds(0, 16)]
  lanes = lax.iota(jnp.int32, 16)

  @pl.loop(0, RPT32)
  def _(j):
    cs = pltpu.async_copy(tab128.at[idx_src.at[j]], rows_s, sem)
    cd = pltpu.async_copy(tab128.at[idx_dst.at[j]], rows_d, sem)
    cs.wait()
    cd.wait()
    for g in range(B // 16):
      rv = lanes + g * 16
      for h in range(4):
        el = plsc.load_gather(rows_s, [rv, jnp.full((16,), h, jnp.int32)])
        er = plsc.load_gather(rows_d, [rv, jnp.full((16,), 4 + h, jnp.int32)])
        e = el + er
        e = jnp.where(e >= 0, e, 0.2 * e) - sh
        ee = jnp.exp(e)
        plsc.store_scatter(rows_s, [rv, jnp.full((16,), h, jnp.int32)], ee)
        plsc.store_scatter(ee_flat, [rv * 16 + h], ee)
    pltpu.sync_copy(rows_s, acc.at[idx_dst.at[j]], add=True)
    pltpu.sync_copy(ee_flat, ee_out.at[base + j])
  plsc.subcore_barrier()

  @pl.when(s < WTILES)
  def _():
    pltpu.sync_copy(acc.at[pl.ds(s * WROWS, WROWS)],
                    esum_out.at[c, pl.ds(s * WROWS, WROWS)])


def sc_gat_edge(tab128, src2d, dst2d, shift):
  mesh = plsc.VectorSubcoreMesh(core_axis_name="c", subcore_axis_name="s")
  fn = pl.kernel(
      _gat_edge_body,
      out_type=(jax.ShapeDtypeStruct((R, B * 16), jnp.float32),
                jax.ShapeDtypeStruct((NCORE, N, CH), jnp.float32)),
      mesh=mesh,
      compiler_params=_SC_PARAMS,
      scratch_types=[
          pltpu.VMEM((RPT32, B), jnp.int32),
          pltpu.VMEM((RPT32, B), jnp.int32),
          pltpu.VMEM((B, CH), jnp.float32),
          pltpu.VMEM((B, CH), jnp.float32),
          pltpu.VMEM((B * 16,), jnp.float32),
          pltpu.VMEM((CH,), jnp.float32),
          pltpu.VMEM_SHARED((NP, CH), jnp.float32),
          pltpu.SemaphoreType.DMA,
      ],
  )
  return fn(tab128, src2d, dst2d, shift)


def _gat_agg_body(feat, ee3, src2d, dst2d, out,
                  idx_src, idx_dst, rows, ee_buf, zbuf, acc, sem):
  c = lax.axis_index("c")
  s = lax.axis_index("s")
  _sc2d_zero(zbuf)
  pltpu.sync_copy(src2d.at[pl.ds(s * RPT, RPT)], idx_src)
  pltpu.sync_copy(dst2d.at[pl.ds(s * RPT, RPT)], idx_dst)

  _coop_zero(s, acc, zbuf)
  plsc.subcore_barrier()

  h0 = 2 * c
  lanes = lax.iota(jnp.int32, 16)
  @pl.loop(0, RPT)
  def _(j):
    cg = pltpu.async_copy(feat.at[idx_src.at[j], pl.ds(c * CH, CH)],
                          rows, sem)
    pltpu.sync_copy(ee3.at[s * RPT + j], ee_buf)
    cg.wait()
    @pl.loop(0, B)
    def _(r):
      rv = jnp.full((16,), r, jnp.int32)
      m0 = plsc.load_gather(ee_buf, [jnp.full((16,), r * 16 + h0, jnp.int32)])
      m1 = plsc.load_gather(ee_buf,
                            [jnp.full((16,), r * 16 + h0 + 1, jnp.int32)])
      for q in range(8):
        m = m0 if q < 4 else m1
        cv = lanes + q * 16
        x = plsc.load_gather(rows, [rv, cv])
        plsc.store_scatter(rows, [rv, cv], x * m)
    pltpu.sync_copy(rows, acc.at[idx_dst.at[j]], add=True)
  plsc.subcore_barrier()

  @pl.when(s < WTILES)
  def _():
    pltpu.sync_copy(acc.at[pl.ds(s * WROWS, WROWS)],
                    out.at[pl.ds(s * WROWS, WROWS), pl.ds(c * CH, CH)])


def sc_gat_aggregate(feat, ee3, src2d, dst2d):
  mesh = plsc.VectorSubcoreMesh(core_axis_name="c", subcore_axis_name="s")
  fn = pl.kernel(
      _gat_agg_body,
      out_type=jax.ShapeDtypeStruct((N, 2 * CH), jnp.float32),
      mesh=mesh,
      compiler_params=_SC_PARAMS,
      scratch_types=[
          pltpu.VMEM((RPT, B), jnp.int32),
          pltpu.VMEM((RPT, B), jnp.int32),
          pltpu.VMEM((B, CH), jnp.float32),
          pltpu.VMEM((B * 16,), jnp.float32),
          pltpu.VMEM((ZR, CH), jnp.float32),
          pltpu.VMEM_SHARED((NP, CH), jnp.float32),
          pltpu.SemaphoreType.DMA,
      ],
  )
  return fn(feat, ee3, src2d, dst2d)


def _mlp1_body(h_ref, agg_ref, deg0_ref, deg1_ref, eps_ref, w0_ref, b0_ref,
               y_ref, stats_ref):
  i = pl.program_id(0)
  deg = jnp.maximum(deg0_ref[...] + deg1_ref[...], 1.0)
  x = h_ref[...] * (1.0 + eps_ref[0]) + agg_ref[...] / deg
  y = jnp.dot(x, w0_ref[...], preferred_element_type=jnp.float32)
  y = jnp.maximum(y + b0_ref[...], 0.0)
  y_ref[...] = y
  s0 = jnp.sum(y, axis=0, keepdims=True)
  s1 = jnp.sum(y * y, axis=0, keepdims=True)
  st = jnp.concatenate([s0, s1], axis=0)
  @pl.when(i == 0)
  def _():
    stats_ref[...] = st
  @pl.when(i > 0)
  def _():
    stats_ref[...] = stats_ref[...] + st


def tc_mlp1(h, agg, deg0, deg1, eps, w0, b0):
  din = h.shape[1]
  dout = w0.shape[1]
  return pl.pallas_call(
      _mlp1_body,
      grid=(GRID,),
      in_specs=[
          pl.BlockSpec((BN, din), lambda i: (i, 0)),
          pl.BlockSpec((BN, din), lambda i: (i, 0)),
          pl.BlockSpec((BN, 1), lambda i: (i, 0)),
          pl.BlockSpec((BN, 1), lambda i: (i, 0)),
          pl.BlockSpec(memory_space=pltpu.SMEM),
          pl.BlockSpec((din, dout), lambda i: (0, 0)),
          pl.BlockSpec((1, dout), lambda i: (0, 0)),
      ],
      out_specs=[
          pl.BlockSpec((BN, dout), lambda i: (i, 0)),
          pl.BlockSpec((2, dout), lambda i: (0, 0)),
      ],
      out_shape=[
          jax.ShapeDtypeStruct((N, dout), jnp.float32),
          jax.ShapeDtypeStruct((2, dout), jnp.float32),
      ],
      compiler_params=pltpu.CompilerParams(
          dimension_semantics=("arbitrary",)),
  )(h, agg, deg0, deg1, eps, w0, b0)


def _mlp2_body(y_ref, stats_ref, gamma_ref, beta_ref, w1_ref, b1_ref, out_ref):
  mu = stats_ref[0:1, :] / N
  var = stats_ref[1:2, :] / N - mu * mu
  scale = gamma_ref[...] * lax.rsqrt(var + 1e-5)
  yhat = (y_ref[...] - mu) * scale + beta_ref[...]
  z = jnp.dot(yhat, w1_ref[...], preferred_element_type=jnp.float32)
  out_ref[...] = jnp.maximum(z + b1_ref[...], 0.0)


def tc_mlp2(y, stats, gamma, beta, w1, b1):
  d = y.shape[1]
  dout = w1.shape[1]
  return pl.pallas_call(
      _mlp2_body,
      grid=(GRID,),
      in_specs=[
          pl.BlockSpec((BN, d), lambda i: (i, 0)),
          pl.BlockSpec((2, d), lambda i: (0, 0)),
          pl.BlockSpec((1, d), lambda i: (0, 0)),
          pl.BlockSpec((1, d), lambda i: (0, 0)),
          pl.BlockSpec((d, dout), lambda i: (0, 0)),
          pl.BlockSpec((1, dout), lambda i: (0, 0)),
      ],
      out_specs=pl.BlockSpec((BN, dout), lambda i: (i, 0)),
      out_shape=jax.ShapeDtypeStruct((N, dout), jnp.float32),
  )(y, stats, gamma, beta, w1, b1)


def _gat_prep_body(h_ref, fc_ref, alr_ref, feat_ref, tab_ref, shift_ref,
                   mx_ref):
  i = pl.program_id(0)
  feat = jnp.dot(h_ref[...], fc_ref[...], preferred_element_type=jnp.float32)
  feat_ref[...] = feat
  tab = jnp.dot(feat, alr_ref[...], preferred_element_type=jnp.float32)
  tab_ref[...] = tab
  ml = jnp.max(tab[:, 0:4])
  mr = jnp.max(tab[:, 4:8])
  @pl.when(i == 0)
  def _():
    mx_ref[0] = ml
    mx_ref[1] = mr
  @pl.when(i > 0)
  def _():
    mx_ref[0] = jnp.maximum(mx_ref[0], ml)
    mx_ref[1] = jnp.maximum(mx_ref[1], mr)
  @pl.when(i == GRID - 1)
  def _():
    shift_ref[...] = jnp.full(
        (8, 128), jnp.maximum(mx_ref[0] + mx_ref[1], 0.0), jnp.float32)


def tc_gat_prep(h, fc, alr_mat):
  d = h.shape[1]
  f = fc.shape[1]
  return pl.pallas_call(
      _gat_prep_body,
      grid=(GRID,),
      in_specs=[
          pl.BlockSpec((BN, d), lambda i: (i, 0)),
          pl.BlockSpec((d, f), lambda i: (0, 0)),
          pl.BlockSpec((f, 128), lambda i: (0, 0)),
      ],
      out_specs=[
          pl.BlockSpec((BN, f), lambda i: (i, 0)),
          pl.BlockSpec((BN, 128), lambda i: (i, 0)),
          pl.BlockSpec((8, 128), lambda i: (0, 0)),
      ],
      out_shape=[
          jax.ShapeDtypeStruct((N, f), jnp.float32),
          jax.ShapeDtypeStruct((N, 128), jnp.float32),
          jax.ShapeDtypeStruct((8, 128), jnp.float32),
      ],
      scratch_shapes=[pltpu.SMEM((2,), jnp.float32)],
      compiler_params=pltpu.CompilerParams(
          dimension_semantics=("arbitrary",)),
  )(h, fc, alr_mat)


def _final_body(h1_ref, h2_ref, h3_ref, rst_ref, es0_ref, es1_ref, bias_ref,
                kmat_ref, mmat_ref, out_ref):
  esum = jnp.maximum(es0_ref[...] + es1_ref[...], 1e-30)
  div = jnp.dot(esum, kmat_ref[...], preferred_element_type=jnp.float32)
  res = rst_ref[...] / div + bias_ref[...]
  res = jnp.maximum(res, 0.0)
  gat = jnp.dot(res, mmat_ref[...], preferred_element_type=jnp.float32)
  out_ref[:, 0:512] = h1_ref[...]
  out_ref[:, 512:1024] = h2_ref[...]
  out_ref[:, 1024:1536] = h3_ref[...]
  out_ref[:, 1536:1600] = gat


def tc_finalize(h1, h2, h3, rst, es0, es1, bias_row, kmat, mmat):
  return pl.pallas_call(
      _final_body,
      grid=(GRID,),
      in_specs=[
          pl.BlockSpec((BN, 512), lambda i: (i, 0)),
          pl.BlockSpec((BN, 512), lambda i: (i, 0)),
          pl.BlockSpec((BN, 512), lambda i: (i, 0)),
          pl.BlockSpec((BN, 256), lambda i: (i, 0)),
          pl.BlockSpec((BN, 4), lambda i: (i, 0)),
          pl.BlockSpec((BN, 4), lambda i: (i, 0)),
          pl.BlockSpec((1, 256), lambda i: (0, 0)),
          pl.BlockSpec((4, 256), lambda i: (0, 0)),
          pl.BlockSpec((256, 64), lambda i: (0, 0)),
      ],
      out_specs=pl.BlockSpec((BN, 1600), lambda i: (i, 0)),
      out_shape=jax.ShapeDtypeStruct((N, 1600), jnp.float32),
  )(h1, h2, h3, rst, es0, es1, bias_row, kmat, mmat)


def kernel(h, edge_index, params):
  src = edge_index[0]
  dst = edge_index[1]
  pad = EP - E
  src2d = jnp.concatenate([src, jnp.zeros((pad,), jnp.int32)]).reshape(R, B)
  dst2d = jnp.concatenate([dst, jnp.full((pad,), N, jnp.int32)]).reshape(R, B)

  deg_parts = sc_degree(dst2d).reshape(NCORE, ND)
  deg0 = deg_parts[0, :N].reshape(N, 1)
  deg1 = deg_parts[1, :N].reshape(N, 1)

  x = h
  outs = []
  for p in params['gins']:
    agg = sc_segment_sum(x, src2d, dst2d)
    eps = p['eps'].reshape(1)
    y, stats = tc_mlp1(x, agg, deg0, deg1, eps, p['W0'], p['b0'].reshape(1, -1))
    x = tc_mlp2(y, stats, p['gamma'].reshape(1, -1), p['beta'].reshape(1, -1),
                p['W1'], p['b1'].reshape(1, -1))
    outs.append(x)

  g = params['gat']
  heads, dh = g['attn_l'].shape
  f = heads * dh
  eye_l = jnp.eye(heads, 128, dtype=jnp.float32)
  eye_r = jnp.eye(heads, 128, k=heads, dtype=jnp.float32)
  alr = (eye_l[:, None, :] * g['attn_l'][:, :, None]
         + eye_r[:, None, :] * g['attn_r'][:, :, None]).reshape(f, 128)

  feat, tab128, shift_arr = tc_gat_prep(x, g['fc'], alr)
  shift = shift_arr[0]
  ee3, esum_parts = sc_gat_edge(tab128, src2d, dst2d, shift)
  rst = sc_gat_aggregate(feat, ee3, src2d, dst2d)

  es0 = esum_parts[0, :, 0:4]
  es1 = esum_parts[1, :, 0:4]

  eye = jnp.eye(heads, dtype=jnp.float32)
  kmat = jnp.repeat(eye, dh, axis=1).astype(jnp.float32)
  mmat = jnp.kron(jnp.ones((heads, 1), jnp.float32),
                  jnp.eye(dh, dtype=jnp.float32)) / heads
  bias_row = g['bias'].reshape(1, f)

  return tc_finalize(outs[0], outs[1], outs[2], rst, es0, es1,
                     bias_row, kmat, mmat)
